# baseline (speedup 1.0000x reference)
import functools

import jax
import jax.numpy as jnp
from jax import lax
from jax.experimental import pallas as pl
from jax.experimental.pallas import tpu as pltpu
from jax.experimental.pallas import tpu_sc as plsc

_NC = 2
_NS = 16
_BLK = 1000


def _dot_t(a, b):
    return lax.dot_general(a, b, (((1,), (1,)), ((), ())),
                           preferred_element_type=jnp.float32)


def _mlp_tc(x, w1, b1, w2, b2):
    n, din = x.shape
    hid = w1.shape[0]
    dout = w2.shape[0]

    def body(x_ref, w1_ref, b1_ref, w2_ref, b2_ref, o_ref):
        h = jnp.maximum(_dot_t(x_ref[...], w1_ref[...]) + b1_ref[...], 0.0)
        o_ref[...] = jnp.maximum(_dot_t(h, w2_ref[...]) + b2_ref[...], 0.0)

    return pl.pallas_call(
        body,
        grid=(n // _BLK,),
        in_specs=[
            pl.BlockSpec((_BLK, din), lambda i: (i, 0)),
            pl.BlockSpec((hid, din), lambda i: (0, 0)),
            pl.BlockSpec((1, hid), lambda i: (0, 0)),
            pl.BlockSpec((dout, hid), lambda i: (0, 0)),
            pl.BlockSpec((1, dout), lambda i: (0, 0)),
        ],
        out_specs=pl.BlockSpec((_BLK, dout), lambda i: (i, 0)),
        out_shape=jax.ShapeDtypeStruct((n, dout), jnp.float32),
    )(x, w1, b1, w2, b2)


def _update_tc(p0, p1, x, w1a, w1b, b1, w2, b2):
    n, d = x.shape
    hid = w1a.shape[0]
    dout = w2.shape[0]

    def body(p0_ref, p1_ref, x_ref, w1a_ref, w1b_ref, b1_ref, w2_ref,
             b2_ref, o_ref):
        aggr = p0_ref[...] + p1_ref[...]
        h = (_dot_t(aggr, w1a_ref[...]) + _dot_t(x_ref[...], w1b_ref[...])
             + b1_ref[...])
        h = jnp.maximum(h, 0.0)
        o_ref[...] = jnp.maximum(_dot_t(h, w2_ref[...]) + b2_ref[...], 0.0)

    return pl.pallas_call(
        body,
        grid=(n // _BLK,),
        in_specs=[
            pl.BlockSpec((_BLK, d), lambda i: (i, 0)),
            pl.BlockSpec((_BLK, d), lambda i: (i, 0)),
            pl.BlockSpec((_BLK, d), lambda i: (i, 0)),
            pl.BlockSpec((hid, d), lambda i: (0, 0)),
            pl.BlockSpec((hid, d), lambda i: (0, 0)),
            pl.BlockSpec((1, hid), lambda i: (0, 0)),
            pl.BlockSpec((dout, hid), lambda i: (0, 0)),
            pl.BlockSpec((1, dout), lambda i: (0, 0)),
        ],
        out_specs=pl.BlockSpec((_BLK, dout), lambda i: (i, 0)),
        out_shape=jax.ShapeDtypeStruct((n, dout), jnp.float32),
    )(p0, p1, x, w1a, w1b, b1, w2, b2)


def _edge_aggregate(msg, idx, zeros, n_pad):
    n, d = msg.shape
    nw, n_chunks, _, chunk = idx.shape
    rows_per_s = n_pad // _NS

    mesh = plsc.VectorSubcoreMesh(core_axis_name="c", subcore_axis_name="s",
                                  num_cores=_NC, num_subcores=_NS)

    @functools.partial(
        pl.kernel,
        mesh=mesh,
        out_type=[jax.ShapeDtypeStruct((n_pad, d), jnp.float32),
                  jax.ShapeDtypeStruct((n_pad, d), jnp.float32)],
        scratch_types=[
            pltpu.VMEM((5, 2, chunk), jnp.int32),
            pltpu.VMEM((chunk, d), jnp.float32),
            pltpu.VMEM((chunk, d), jnp.float32),
            pltpu.VMEM((chunk, d), jnp.float32),
            pltpu.VMEM((chunk, d), jnp.float32),
            pltpu.VMEM((chunk, d), jnp.float32),
            pltpu.VMEM_SHARED((n_pad, d), jnp.float32),
            (pltpu.SemaphoreType.DMA,) * 5,
            (pltpu.SemaphoreType.DMA,) * 5,
            (pltpu.SemaphoreType.DMA,) * 5,
        ],
    )
    def body(msg_hbm, idx_hbm, zero_hbm, out0_hbm, out1_hbm,
             ridx, rows0, rows1, rows2, rows3, rows4, acc, idsem, gsem, ssem):
        c = lax.axis_index("c")
        s = lax.axis_index("s")
        wid = s * _NC + c
        r0 = s * rows_per_s
        rows = (rows0, rows1, rows2, rows3, rows4)
        pltpu.sync_copy(zero_hbm.at[pl.ds(r0, rows_per_s)],
                        acc.at[pl.ds(r0, rows_per_s)])
        plsc.subcore_barrier()

        def idx_load(g, b):
            pltpu.async_copy(idx_hbm.at[wid, g], ridx.at[b], idsem[b])

        def idx_wait(g, b):
            pltpu.make_async_copy(idx_hbm.at[wid, g], ridx.at[b],
                                  idsem[b]).wait()

        def gather(g, b):
            pltpu.async_copy(msg_hbm.at[ridx.at[b, 0]], rows[b], gsem[b])

        def gather_wait(g, b):
            pltpu.make_async_copy(msg_hbm.at[ridx.at[b, 0]], rows[b],
                                  gsem[b]).wait()

        def scatter(g, b):
            pltpu.async_copy(rows[b], acc.at[ridx.at[b, 1]], ssem[b], add=True)

        def scatter_wait(b):
            pltpu.make_async_copy(rows[b], acc.at[ridx.at[b, 1]], ssem[b]).wait()

        def stage(g, b):
            gather_wait(g, b)
            scatter(g, b)

            @pl.when(g + 3 < n_chunks)
            def _():
                scatter_wait((b + 3) % 5)
                idx_load(g + 3, (b + 3) % 5)

            @pl.when(g + 2 < n_chunks)
            def _():
                idx_wait(g + 2, (b + 2) % 5)
                gather(g + 2, (b + 2) % 5)

        idx_load(0, 0)
        idx_load(1, 1)
        idx_load(2, 2)
        idx_wait(0, 0)
        gather(0, 0)
        idx_wait(1, 1)
        gather(1, 1)
        gather_wait(0, 0)
        scatter(0, 0)
        idx_load(3, 3)
        idx_wait(2, 2)
        gather(2, 2)
        gather_wait(1, 1)
        scatter(1, 1)
        idx_load(4, 4)
        idx_wait(3, 3)
        gather(3, 3)

        def group(j, carry):
            for k in range(5):
                stage(2 + 5 * j + k, (2 + k) % 5)
            return carry

        n_main = (n_chunks - 2) // 5
        lax.fori_loop(0, n_main, group, 0)
        for g in range(2 + 5 * n_main, n_chunks):
            gather_wait(g, g % 5)
            scatter(g, g % 5)
            if g + 3 < n_chunks:
                scatter_wait((g + 3) % 5)
                idx_load(g + 3, (g + 3) % 5)
            if g + 2 < n_chunks:
                idx_wait(g + 2, (g + 2) % 5)
                gather(g + 2, (g + 2) % 5)
        for b in range(5):
            scatter_wait(b)
        plsc.subcore_barrier()

        @pl.when(c == 0)
        def _():
            pltpu.sync_copy(acc.at[pl.ds(r0, rows_per_s)],
                            out0_hbm.at[pl.ds(r0, rows_per_s)])

        @pl.when(c == 1)
        def _():
            pltpu.sync_copy(acc.at[pl.ds(r0, rows_per_s)],
                            out1_hbm.at[pl.ds(r0, rows_per_s)])

    return body(msg, idx, zeros)


def kernel(x, edge_index, params):
    n, d = x.shape
    nw = _NC * _NS
    e = edge_index.shape[1]
    chunk = 72
    per_w = e // nw
    n_chunks = -(-per_w // chunk)
    pad = n_chunks * chunk - per_w
    n_pad = -(-(n + 1) // (8 * _NS)) * (8 * _NS)
    src_w = jnp.pad(edge_index[0].astype(jnp.int32).reshape(nw, per_w),
                    ((0, 0), (0, pad)))
    dst_w = jnp.pad(edge_index[1].astype(jnp.int32).reshape(nw, per_w),
                    ((0, 0), (0, pad)), constant_values=n)
    idx = jnp.stack([src_w.reshape(nw, n_chunks, chunk),
                     dst_w.reshape(nw, n_chunks, chunk)], axis=2)
    zeros = jnp.zeros((n_pad, d), jnp.float32)
    for p in params:
        m, u = p['mlp'], p['update']
        msg = _mlp_tc(x, m['W1'], m['b1'].reshape(1, -1),
                      m['W2'], m['b2'].reshape(1, -1))
        p0, p1 = _edge_aggregate(msg, idx, zeros, n_pad)
        x = _update_tc(p0, p1, x,
                       u['W1'][:, :d], u['W1'][:, d:],
                       u['b1'].reshape(1, -1), u['W2'],
                       u['b2'].reshape(1, -1))
    return x


# speedup vs baseline: 1.0802x; 1.0802x over previous
import functools

import jax
import jax.numpy as jnp
from jax import lax
from jax.experimental import pallas as pl
from jax.experimental.pallas import tpu as pltpu
from jax.experimental.pallas import tpu_sc as plsc

_NC = 2
_NS = 16
_BLK = 1000


def _dot_t(a, b):
    return lax.dot_general(a, b, (((1,), (1,)), ((), ())),
                           preferred_element_type=jnp.float32)


def _mlp_tc(x, w1, b1, w2, b2):
    n, din = x.shape
    hid = w1.shape[0]
    dout = w2.shape[0]

    def body(x_ref, w1_ref, b1_ref, w2_ref, b2_ref, o_ref):
        h = jnp.maximum(_dot_t(x_ref[...], w1_ref[...]) + b1_ref[...], 0.0)
        o_ref[...] = jnp.maximum(_dot_t(h, w2_ref[...]) + b2_ref[...], 0.0)

    return pl.pallas_call(
        body,
        grid=(n // _BLK,),
        in_specs=[
            pl.BlockSpec((_BLK, din), lambda i: (i, 0)),
            pl.BlockSpec((hid, din), lambda i: (0, 0)),
            pl.BlockSpec((1, hid), lambda i: (0, 0)),
            pl.BlockSpec((dout, hid), lambda i: (0, 0)),
            pl.BlockSpec((1, dout), lambda i: (0, 0)),
        ],
        out_specs=pl.BlockSpec((_BLK, dout), lambda i: (i, 0)),
        out_shape=jax.ShapeDtypeStruct((n, dout), jnp.float32),
    )(x, w1, b1, w2, b2)


def _update_tc(p0, p1, x, w1a, w1b, b1, w2, b2):
    n, d = x.shape
    hid = w1a.shape[0]
    dout = w2.shape[0]

    def body(p0_ref, p1_ref, x_ref, w1a_ref, w1b_ref, b1_ref, w2_ref,
             b2_ref, o_ref):
        aggr = p0_ref[...] + p1_ref[...]
        h = (_dot_t(aggr, w1a_ref[...]) + _dot_t(x_ref[...], w1b_ref[...])
             + b1_ref[...])
        h = jnp.maximum(h, 0.0)
        o_ref[...] = jnp.maximum(_dot_t(h, w2_ref[...]) + b2_ref[...], 0.0)

    return pl.pallas_call(
        body,
        grid=(n // _BLK,),
        in_specs=[
            pl.BlockSpec((_BLK, d), lambda i: (i, 0)),
            pl.BlockSpec((_BLK, d), lambda i: (i, 0)),
            pl.BlockSpec((_BLK, d), lambda i: (i, 0)),
            pl.BlockSpec((hid, d), lambda i: (0, 0)),
            pl.BlockSpec((hid, d), lambda i: (0, 0)),
            pl.BlockSpec((1, hid), lambda i: (0, 0)),
            pl.BlockSpec((dout, hid), lambda i: (0, 0)),
            pl.BlockSpec((1, dout), lambda i: (0, 0)),
        ],
        out_specs=pl.BlockSpec((_BLK, dout), lambda i: (i, 0)),
        out_shape=jax.ShapeDtypeStruct((n, dout), jnp.float32),
    )(p0, p1, x, w1a, w1b, b1, w2, b2)


def _update_msg_tc(p0, p1, x, w1a, w1b, b1, w2, b2, mw1, mb1, mw2, mb2):
    n, d = x.shape
    hid = w1a.shape[0]
    dout = w2.shape[0]

    def body(p0_ref, p1_ref, x_ref, w1a_ref, w1b_ref, b1_ref, w2_ref,
             b2_ref, mw1_ref, mb1_ref, mw2_ref, mb2_ref, xo_ref, mo_ref):
        aggr = p0_ref[...] + p1_ref[...]
        h = (_dot_t(aggr, w1a_ref[...]) + _dot_t(x_ref[...], w1b_ref[...])
             + b1_ref[...])
        h = jnp.maximum(h, 0.0)
        xn = jnp.maximum(_dot_t(h, w2_ref[...]) + b2_ref[...], 0.0)
        xo_ref[...] = xn
        mh = jnp.maximum(_dot_t(xn, mw1_ref[...]) + mb1_ref[...], 0.0)
        mo_ref[...] = jnp.maximum(_dot_t(mh, mw2_ref[...]) + mb2_ref[...], 0.0)

    return pl.pallas_call(
        body,
        grid=(n // _BLK,),
        in_specs=[
            pl.BlockSpec((_BLK, d), lambda i: (i, 0)),
            pl.BlockSpec((_BLK, d), lambda i: (i, 0)),
            pl.BlockSpec((_BLK, d), lambda i: (i, 0)),
            pl.BlockSpec((hid, d), lambda i: (0, 0)),
            pl.BlockSpec((hid, d), lambda i: (0, 0)),
            pl.BlockSpec((1, hid), lambda i: (0, 0)),
            pl.BlockSpec((dout, hid), lambda i: (0, 0)),
            pl.BlockSpec((1, dout), lambda i: (0, 0)),
            pl.BlockSpec((hid, d), lambda i: (0, 0)),
            pl.BlockSpec((1, hid), lambda i: (0, 0)),
            pl.BlockSpec((dout, hid), lambda i: (0, 0)),
            pl.BlockSpec((1, dout), lambda i: (0, 0)),
        ],
        out_specs=[pl.BlockSpec((_BLK, dout), lambda i: (i, 0)),
                   pl.BlockSpec((_BLK, dout), lambda i: (i, 0))],
        out_shape=[jax.ShapeDtypeStruct((n, dout), jnp.float32),
                   jax.ShapeDtypeStruct((n, dout), jnp.float32)],
    )(p0, p1, x, w1a, w1b, b1, w2, b2, mw1, mb1, mw2, mb2)


def _edge_aggregate(msg, src, dst, zeros, n_pad):
    n, d = msg.shape
    nw, n_chunks, chunk = dst.shape
    per_w = n_chunks * chunk
    rows_per_s = n_pad // _NS

    mesh = plsc.VectorSubcoreMesh(core_axis_name="c", subcore_axis_name="s",
                                  num_cores=_NC, num_subcores=_NS)

    @functools.partial(
        pl.kernel,
        mesh=mesh,
        out_type=[jax.ShapeDtypeStruct((n_pad, d), jnp.float32),
                  jax.ShapeDtypeStruct((n_pad, d), jnp.float32)],
        scratch_types=[
            pltpu.VMEM((n_chunks * chunk,), jnp.int32),
            pltpu.VMEM((3, chunk), jnp.int32),
            pltpu.VMEM((chunk, d), jnp.float32),
            pltpu.VMEM((chunk, d), jnp.float32),
            pltpu.VMEM((chunk, d), jnp.float32),
            pltpu.VMEM_SHARED((n_pad, d), jnp.float32),
            pltpu.SemaphoreType.DMA,
            (pltpu.SemaphoreType.DMA,) * 3,
            (pltpu.SemaphoreType.DMA,) * 3,
            (pltpu.SemaphoreType.DMA,) * 3,
        ],
    )
    def body(msg_hbm, src_flat_hbm, dst_hbm, zero_hbm, out0_hbm, out1_hbm,
             sidx, didx, rows0, rows1, rows2, acc, isem, idsem, gsem, ssem):
        c = lax.axis_index("c")
        s = lax.axis_index("s")
        wid = s * _NC + c
        r0 = s * rows_per_s
        rows = (rows0, rows1, rows2)
        icp = pltpu.async_copy(src_flat_hbm.at[pl.ds(wid * per_w, per_w)],
                               sidx, isem)
        pltpu.sync_copy(zero_hbm.at[pl.ds(r0, rows_per_s)],
                        acc.at[pl.ds(r0, rows_per_s)])
        icp.wait()
        plsc.subcore_barrier()

        def didx_load(g, b):
            pltpu.async_copy(dst_hbm.at[wid, g], didx.at[b], idsem[b])

        def didx_wait(g, b):
            pltpu.make_async_copy(dst_hbm.at[wid, g], didx.at[b],
                                  idsem[b]).wait()

        def gather(g, b):
            pltpu.async_copy(msg_hbm.at[sidx.at[pl.ds(g * chunk, chunk)]],
                             rows[b], gsem[b])

        def gather_wait(g, b):
            pltpu.make_async_copy(msg_hbm.at[sidx.at[pl.ds(g * chunk, chunk)]],
                                  rows[b], gsem[b]).wait()

        def scatter(g, b):
            pltpu.async_copy(rows[b], acc.at[didx.at[b]], ssem[b], add=True)

        def scatter_wait(b):
            pltpu.make_async_copy(rows[b], acc.at[didx.at[b]], ssem[b]).wait()

        def stage(g, b, bn):
            gather_wait(g, b)
            didx_wait(g, b)
            scatter(g, b)

            @pl.when(g + 2 < n_chunks)
            def _():
                scatter_wait(bn)
                didx_load(g + 2, bn)
                gather(g + 2, bn)

        didx_load(0, 0)
        didx_load(1, 1)
        gather(0, 0)
        gather(1, 1)
        gather_wait(0, 0)
        didx_wait(0, 0)
        scatter(0, 0)
        didx_load(2, 2)
        gather(2, 2)
        stage(1, 1, 0)

        def triple(j, carry):
            for k in (0, 1, 2):
                g = 2 + 3 * j + k
                stage(g, (2 + k) % 3, (4 + k) % 3)
            return carry

        lax.fori_loop(0, (n_chunks - 2) // 3, triple, 0)
        for g in range(n_chunks - (n_chunks - 2) % 3, n_chunks):
            stage(g, g % 3, (g + 2) % 3)
        scatter_wait((n_chunks - 3) % 3)
        scatter_wait((n_chunks - 2) % 3)
        scatter_wait((n_chunks - 1) % 3)
        plsc.subcore_barrier()

        @pl.when(c == 0)
        def _():
            pltpu.sync_copy(acc.at[pl.ds(r0, rows_per_s)],
                            out0_hbm.at[pl.ds(r0, rows_per_s)])

        @pl.when(c == 1)
        def _():
            pltpu.sync_copy(acc.at[pl.ds(r0, rows_per_s)],
                            out1_hbm.at[pl.ds(r0, rows_per_s)])

    return body(msg, src, dst, zeros)


def kernel(x, edge_index, params):
    n, d = x.shape
    nw = _NC * _NS
    e = edge_index.shape[1]
    chunk = 80
    src = edge_index[0].astype(jnp.int32)
    dst = edge_index[1].astype(jnp.int32).reshape(nw, e // (nw * chunk), chunk)
    n_pad = ((n + 8 * _NS - 1) // (8 * _NS)) * (8 * _NS)
    zeros = jnp.zeros((n_pad, d), jnp.float32)

    m0 = params[0]['mlp']
    msg = _mlp_tc(x, m0['W1'], m0['b1'].reshape(1, -1),
                  m0['W2'], m0['b2'].reshape(1, -1))
    for layer, p in enumerate(params):
        u = p['update']
        p0, p1 = _edge_aggregate(msg, src, dst, zeros, n_pad)
        if layer + 1 < len(params):
            mn = params[layer + 1]['mlp']
            x, msg = _update_msg_tc(p0, p1, x,
                                    u['W1'][:, :d], u['W1'][:, d:],
                                    u['b1'].reshape(1, -1), u['W2'],
                                    u['b2'].reshape(1, -1),
                                    mn['W1'], mn['b1'].reshape(1, -1),
                                    mn['W2'], mn['b2'].reshape(1, -1))
        else:
            x = _update_tc(p0, p1, x,
                           u['W1'][:, :d], u['W1'][:, d:],
                           u['b1'].reshape(1, -1), u['W2'],
                           u['b2'].reshape(1, -1))
    return x


# speedup vs baseline: 1.1086x; 1.0263x over previous
import functools

import jax
import jax.numpy as jnp
from jax import lax
from jax.experimental import pallas as pl
from jax.experimental.pallas import tpu as pltpu
from jax.experimental.pallas import tpu_sc as plsc

_NC = 2
_NS = 16
_BLK = 1000


def _dot_t(a, b):
    return lax.dot_general(a, b, (((1,), (1,)), ((), ())),
                           preferred_element_type=jnp.float32)


def _mlp_tc(x, w1, b1, w2, b2):
    n, din = x.shape
    hid = w1.shape[0]
    dout = w2.shape[0]

    def body(x_ref, w1_ref, b1_ref, w2_ref, b2_ref, o_ref):
        h = jnp.maximum(_dot_t(x_ref[...], w1_ref[...]) + b1_ref[...], 0.0)
        o_ref[...] = jnp.maximum(_dot_t(h, w2_ref[...]) + b2_ref[...], 0.0)

    return pl.pallas_call(
        body,
        grid=(n // _BLK,),
        in_specs=[
            pl.BlockSpec((_BLK, din), lambda i: (i, 0)),
            pl.BlockSpec((hid, din), lambda i: (0, 0)),
            pl.BlockSpec((1, hid), lambda i: (0, 0)),
            pl.BlockSpec((dout, hid), lambda i: (0, 0)),
            pl.BlockSpec((1, dout), lambda i: (0, 0)),
        ],
        out_specs=pl.BlockSpec((_BLK, dout), lambda i: (i, 0)),
        out_shape=jax.ShapeDtypeStruct((n, dout), jnp.float32),
    )(x, w1, b1, w2, b2)


def _update_tc(p0, p1, x, w1a, w1b, b1, w2, b2):
    n, d = x.shape
    hid = w1a.shape[0]
    dout = w2.shape[0]

    def body(p0_ref, p1_ref, x_ref, w1a_ref, w1b_ref, b1_ref, w2_ref,
             b2_ref, o_ref):
        aggr = p0_ref[...] + p1_ref[...]
        h = (_dot_t(aggr, w1a_ref[...]) + _dot_t(x_ref[...], w1b_ref[...])
             + b1_ref[...])
        h = jnp.maximum(h, 0.0)
        o_ref[...] = jnp.maximum(_dot_t(h, w2_ref[...]) + b2_ref[...], 0.0)

    return pl.pallas_call(
        body,
        grid=(n // _BLK,),
        in_specs=[
            pl.BlockSpec((_BLK, d), lambda i: (i, 0)),
            pl.BlockSpec((_BLK, d), lambda i: (i, 0)),
            pl.BlockSpec((_BLK, d), lambda i: (i, 0)),
            pl.BlockSpec((hid, d), lambda i: (0, 0)),
            pl.BlockSpec((hid, d), lambda i: (0, 0)),
            pl.BlockSpec((1, hid), lambda i: (0, 0)),
            pl.BlockSpec((dout, hid), lambda i: (0, 0)),
            pl.BlockSpec((1, dout), lambda i: (0, 0)),
        ],
        out_specs=pl.BlockSpec((_BLK, dout), lambda i: (i, 0)),
        out_shape=jax.ShapeDtypeStruct((n, dout), jnp.float32),
    )(p0, p1, x, w1a, w1b, b1, w2, b2)


def _edge_aggregate(msg, src, dst, zeros, n_pad):
    n, d = msg.shape
    nw, n_chunks, chunk = dst.shape
    per_w = n_chunks * chunk
    rows_per_s = n_pad // _NS

    mesh = plsc.VectorSubcoreMesh(core_axis_name="c", subcore_axis_name="s",
                                  num_cores=_NC, num_subcores=_NS)

    @functools.partial(
        pl.kernel,
        mesh=mesh,
        out_type=[jax.ShapeDtypeStruct((n_pad, d), jnp.float32),
                  jax.ShapeDtypeStruct((n_pad, d), jnp.float32)],
        scratch_types=[
            pltpu.VMEM((n_chunks * chunk,), jnp.int32),
            pltpu.VMEM((3, chunk), jnp.int32),
            pltpu.VMEM((chunk, d), jnp.float32),
            pltpu.VMEM((chunk, d), jnp.float32),
            pltpu.VMEM((chunk, d), jnp.float32),
            pltpu.VMEM_SHARED((n_pad, d), jnp.float32),
            pltpu.SemaphoreType.DMA,
            (pltpu.SemaphoreType.DMA,) * 3,
            (pltpu.SemaphoreType.DMA,) * 3,
            (pltpu.SemaphoreType.DMA,) * 3,
        ],
    )
    def body(msg_hbm, src_flat_hbm, dst_hbm, zero_hbm, out0_hbm, out1_hbm,
             sidx, didx, rows0, rows1, rows2, acc, isem, idsem, gsem, ssem):
        c = lax.axis_index("c")
        s = lax.axis_index("s")
        wid = s * _NC + c
        r0 = s * rows_per_s
        rows = (rows0, rows1, rows2)
        icp = pltpu.async_copy(src_flat_hbm.at[pl.ds(wid * per_w, per_w)],
                               sidx, isem)
        pltpu.sync_copy(zero_hbm.at[pl.ds(r0, rows_per_s)],
                        acc.at[pl.ds(r0, rows_per_s)])
        icp.wait()
        plsc.subcore_barrier()

        def didx_load(g, b):
            pltpu.async_copy(dst_hbm.at[wid, g], didx.at[b], idsem[b])

        def didx_wait(g, b):
            pltpu.make_async_copy(dst_hbm.at[wid, g], didx.at[b],
                                  idsem[b]).wait()

        def gather(g, b):
            pltpu.async_copy(msg_hbm.at[sidx.at[pl.ds(g * chunk, chunk)]],
                             rows[b], gsem[b])

        def gather_wait(g, b):
            pltpu.make_async_copy(msg_hbm.at[sidx.at[pl.ds(g * chunk, chunk)]],
                                  rows[b], gsem[b]).wait()

        def scatter(g, b):
            pltpu.async_copy(rows[b], acc.at[didx.at[b]], ssem[b], add=True)

        def scatter_wait(b):
            pltpu.make_async_copy(rows[b], acc.at[didx.at[b]], ssem[b]).wait()

        def stage(g, b, bn):
            gather_wait(g, b)
            didx_wait(g, b)
            scatter(g, b)

            @pl.when(g + 2 < n_chunks)
            def _():
                scatter_wait(bn)
                didx_load(g + 2, bn)
                gather(g + 2, bn)

        didx_load(0, 0)
        didx_load(1, 1)
        gather(0, 0)
        gather(1, 1)
        gather_wait(0, 0)
        didx_wait(0, 0)
        scatter(0, 0)
        didx_load(2, 2)
        gather(2, 2)
        stage(1, 1, 0)

        def triple(j, carry):
            for k in (0, 1, 2):
                g = 2 + 3 * j + k
                stage(g, (2 + k) % 3, (4 + k) % 3)
            return carry

        lax.fori_loop(0, (n_chunks - 2) // 3, triple, 0)
        for g in range(n_chunks - (n_chunks - 2) % 3, n_chunks):
            stage(g, g % 3, (g + 2) % 3)
        scatter_wait((n_chunks - 3) % 3)
        scatter_wait((n_chunks - 2) % 3)
        scatter_wait((n_chunks - 1) % 3)
        plsc.subcore_barrier()

        @pl.when(c == 0)
        def _():
            pltpu.sync_copy(acc.at[pl.ds(r0, rows_per_s)],
                            out0_hbm.at[pl.ds(r0, rows_per_s)])

        @pl.when(c == 1)
        def _():
            pltpu.sync_copy(acc.at[pl.ds(r0, rows_per_s)],
                            out1_hbm.at[pl.ds(r0, rows_per_s)])

    return body(msg, src, dst, zeros)


def kernel(x, edge_index, params):
    n, d = x.shape
    nw = _NC * _NS
    e = edge_index.shape[1]
    chunk = 80
    src = edge_index[0].astype(jnp.int32)
    dst = edge_index[1].astype(jnp.int32).reshape(nw, e // (nw * chunk), chunk)
    n_pad = ((n + 8 * _NS - 1) // (8 * _NS)) * (8 * _NS)
    zeros = jnp.zeros((n_pad, d), jnp.float32)

    for p in params:
        m, u = p['mlp'], p['update']
        msg = _mlp_tc(x, m['W1'], m['b1'].reshape(1, -1),
                      m['W2'], m['b2'].reshape(1, -1))
        p0, p1 = _edge_aggregate(msg, src, dst, zeros, n_pad)
        x = _update_tc(p0, p1, x,
                       u['W1'][:, :d], u['W1'][:, d:],
                       u['b1'].reshape(1, -1), u['W2'],
                       u['b2'].reshape(1, -1))
    return x
